# Initial kernel scaffold; baseline (speedup 1.0000x reference)
import jax
import jax.numpy as jnp
from jax.experimental import pallas as pl


def kernel(x, table):
    raise NotImplementedError("write your pallas kernel here")


# speedup vs baseline: 1.4362x; 1.4362x over previous
import functools

import jax
import jax.numpy as jnp
from jax import lax
from jax.experimental import pallas as pl
from jax.experimental.pallas import tpu as pltpu
from jax.experimental.pallas import tpu_sc as plsc

_NUM_CORES = 2
_NUM_SUBCORES = 16
_NW = _NUM_CORES * _NUM_SUBCORES
_CH = 128


@functools.lru_cache(maxsize=None)
def _make_gather(n_rows: int, d: int):
  assert n_rows % (_NW * _CH) == 0
  k = n_rows // (_NW * _CH)
  mesh = plsc.VectorSubcoreMesh(
      core_axis_name="c", subcore_axis_name="s",
      num_cores=_NUM_CORES, num_subcores=_NUM_SUBCORES)

  @functools.partial(
      pl.kernel,
      mesh=mesh,
      out_type=jax.ShapeDtypeStruct((n_rows, d), jnp.float32),
      compiler_params=pltpu.CompilerParams(use_tc_tiling_on_sc=False),
      scratch_types=[
          pltpu.VMEM((k, _CH), jnp.int32),
          pltpu.VMEM((_CH, d), jnp.float32),
          pltpu.SemaphoreType.DMA,
      ],
  )
  def gather_kernel(idx_hbm, table_hbm, out_hbm, idx_v, rows_v, sem):
    wid = lax.axis_index("s") * _NUM_CORES + lax.axis_index("c")
    base = wid * (k * _CH)
    pltpu.sync_copy(idx_hbm.at[pl.ds(wid * k, k)], idx_v)

    def step(j, carry):
      pltpu.async_copy(table_hbm.at[idx_v.at[j]], rows_v, sem).wait()
      pltpu.sync_copy(rows_v, out_hbm.at[pl.ds(base + j * _CH, _CH)])
      return carry

    lax.fori_loop(0, k, step, 0)

  return gather_kernel


def kernel(x, table):
  b, f = x.shape
  v, d = table.shape
  n = b * f
  idx = x.reshape(n // _CH, _CH).astype(jnp.int32)
  out = _make_gather(n, d)(idx, table)
  return out.reshape(b, f, d)


# speedup vs baseline: 1.5766x; 1.0977x over previous
import functools

import jax
import jax.numpy as jnp
from jax import lax
from jax.experimental import pallas as pl
from jax.experimental.pallas import tpu as pltpu
from jax.experimental.pallas import tpu_sc as plsc

_NUM_CORES = 2
_NUM_SUBCORES = 16
_NW = _NUM_CORES * _NUM_SUBCORES
_CH = 128


_M = 4


@functools.lru_cache(maxsize=None)
def _make_gather(n_rows: int, d: int):
  assert n_rows % (_NW * _CH) == 0
  k = n_rows // (_NW * _CH)
  assert k % (2 * _M) == 0
  grp = _M * _CH
  n_steps = k // _M
  mesh = plsc.VectorSubcoreMesh(
      core_axis_name="c", subcore_axis_name="s",
      num_cores=_NUM_CORES, num_subcores=_NUM_SUBCORES)

  @functools.partial(
      pl.kernel,
      mesh=mesh,
      out_type=jax.ShapeDtypeStruct((n_rows, d), jnp.float32),
      compiler_params=pltpu.CompilerParams(use_tc_tiling_on_sc=False),
      scratch_types=[
          pltpu.VMEM((k, _CH), jnp.int32),
          pltpu.VMEM((grp, d), jnp.float32),
          pltpu.VMEM((grp, d), jnp.float32),
          pltpu.SemaphoreType.DMA,
          pltpu.SemaphoreType.DMA,
      ],
  )
  def gather_kernel(idx_hbm, table_hbm, out_hbm, idx_v, buf0, buf1, sem0,
                    sem1):
    wid = lax.axis_index("s") * _NUM_CORES + lax.axis_index("c")
    base = wid * (k * _CH)
    pltpu.sync_copy(idx_hbm.at[pl.ds(wid * k, k)], idx_v)
    bufs = (buf0, buf1)
    sems = (sem0, sem1)

    def fire(s, buf, sem):
      for i in range(_M):
        pltpu.async_copy(table_hbm.at[idx_v.at[s * _M + i]],
                         buf.at[pl.ds(i * _CH, _CH)], sem)

    def drain(buf, sem):
      pltpu.make_async_copy(table_hbm.at[pl.ds(0, grp)], buf, sem).wait()

    fire(0, buf0, sem0)

    @pl.loop(0, n_steps, step=2)
    def _body(s0):
      for b in range(2):
        s = s0 + b
        nxt = 1 - b

        @pl.when(s + 1 < n_steps)
        def _():
          fire(s + 1, bufs[nxt], sems[nxt])

        drain(bufs[b], sems[b])
        pltpu.sync_copy(bufs[b], out_hbm.at[pl.ds(base + s * grp, grp)])

  return gather_kernel


def kernel(x, table):
  b, f = x.shape
  v, d = table.shape
  n = b * f
  idx = x.reshape(n // _CH, _CH).astype(jnp.int32)
  out = _make_gather(n, d)(idx, table)
  return out.reshape(b, f, d)
